# baseline (device time: 5047 ns/iter reference)
import jax
import jax.numpy as jnp
from jax.experimental import pallas as pl
from jax.experimental.pallas import tpu as pltpu

K = 4


def kernel(x):
    m, n = x.shape
    rows = m // K

    def body(x_hbm, out_ref, vmem_ref, sems):
        cps = []
        for i in range(K):
            cp = pltpu.make_async_copy(
                x_hbm.at[pl.ds(i * rows, rows), :],
                vmem_ref.at[pl.ds(i * rows, rows), :],
                sems.at[i],
            )
            cp.start()
            cps.append(cp)
        for cp in cps:
            cp.wait()
        out_ref[...] = jnp.sum(vmem_ref[...], axis=0, keepdims=True)

    return pl.pallas_call(
        body,
        out_shape=jax.ShapeDtypeStruct((1, n), x.dtype),
        in_specs=[pl.BlockSpec(memory_space=pl.ANY)],
        out_specs=pl.BlockSpec(memory_space=pltpu.VMEM),
        scratch_shapes=[
            pltpu.VMEM((m, n), x.dtype),
            pltpu.SemaphoreType.DMA((K,)),
        ],
    )(x)
